# baseline (speedup 1.0000x reference)
import jax
import jax.numpy as jnp
from jax.experimental import pallas as pl


def _body(x_ref, kv_ref, m_ref, o_ref):
    wt = (kv_ref[...] * m_ref[...]).astype(jnp.bfloat16)
    x = x_ref[0]
    o_ref[0] = jax.lax.dot_general(
        x, wt,
        dimension_numbers=(((1,), (1,)), ((), ())),
        preferred_element_type=jnp.float32,
    ).astype(jnp.bfloat16)


def kernel(inputs, kernel_values, mask):
    b, c, h, w = inputs.shape
    f = kernel_values.shape[0]
    hw = h * w
    flat = jnp.reshape(inputs, (c, b * hw))
    xt = jnp.transpose(jnp.reshape(flat, (c, b, hw)), (1, 2, 0))
    xt = xt.astype(jnp.bfloat16)

    z = pl.pallas_call(
        _body,
        grid=(b,),
        in_specs=[
            pl.BlockSpec((1, hw, c), lambda i: (i, 0, 0)),
            pl.BlockSpec((f, c), lambda i: (0, 0)),
            pl.BlockSpec((f, c), lambda i: (0, 0)),
        ],
        out_specs=pl.BlockSpec((1, hw, f), lambda i: (i, 0, 0)),
        out_shape=jax.ShapeDtypeStruct((b, hw, f), jnp.bfloat16),
    )(xt, kernel_values, mask)

    out = jnp.transpose(z, (0, 2, 1)).astype(jnp.float32)
    return jnp.reshape(out, (b, f, h, w))


# speedup vs baseline: 1.1235x; 1.1235x over previous
import jax
import jax.numpy as jnp
from jax.experimental import pallas as pl


def _body(x_ref, kv_ref, m_ref, o_ref):
    wt = (kv_ref[...] * m_ref[...]).astype(jnp.bfloat16)
    x = x_ref[0]
    o_ref[0] = jax.lax.dot_general(
        x, wt,
        dimension_numbers=(((1,), (1,)), ((), ())),
        preferred_element_type=jnp.float32,
    )


def kernel(inputs, kernel_values, mask):
    b, c, h, w = inputs.shape
    f = kernel_values.shape[0]
    hw = h * w
    flat = jnp.reshape(inputs, (c, b * hw))
    xt = jnp.transpose(jnp.reshape(flat, (c, b, hw)), (1, 2, 0))
    xt = xt.astype(jnp.bfloat16)

    z = pl.pallas_call(
        _body,
        grid=(b,),
        in_specs=[
            pl.BlockSpec((1, hw, c), lambda i: (i, 0, 0)),
            pl.BlockSpec((f, c), lambda i: (0, 0)),
            pl.BlockSpec((f, c), lambda i: (0, 0)),
        ],
        out_specs=pl.BlockSpec((1, hw, f), lambda i: (i, 0, 0)),
        out_shape=jax.ShapeDtypeStruct((b, hw, f), jnp.float32),
    )(xt, kernel_values, mask)

    out = jnp.transpose(z, (0, 2, 1))
    return jnp.reshape(out, (b, f, h, w))
